# baseline (device time: 87466 ns/iter reference)
import jax
import jax.numpy as jnp
from jax import lax
from jax.experimental import pallas as pl
from jax.experimental.pallas import tpu as pltpu

N_DEV = 16
NR = 8
NL = 7
B = 2
S = 256
D = 768
HQ = 4
DH = 64
C = HQ * DH
KV = 2 * C
SCALE = 0.125


def kernel(x, Wq, Wk, Wv, Wo):
    def body(x_ref, wq_ref, wk_ref, wv_ref, wo_ref, out_ref,
             kv_all, ssem_r, rsem_r, ssem_l, rsem_l):
        my = lax.axis_index("i")
        left = lax.rem(my + N_DEV - 1, N_DEV)
        right = lax.rem(my + 1, N_DEV)

        barrier = pltpu.get_barrier_semaphore()
        for nbr in (left, right):
            pl.semaphore_signal(barrier, inc=1, device_id=(nbr,),
                                device_id_type=pl.DeviceIdType.MESH)
        pl.semaphore_wait(barrier, 2)

        row = lax.broadcasted_iota(jnp.int32, (S, C), 0).astype(jnp.float32)
        col = lax.broadcasted_iota(jnp.int32, (S, C), 1)
        j = (lax.rem(col, DH) // 2).astype(jnp.float32)
        inv = jnp.exp(j * (-2.0 * jnp.log(10000.0) / DH))
        pos = row + my.astype(jnp.float32) * float(S)
        ang = pos * inv
        cos_t = jnp.cos(ang)
        sin_t = jnp.sin(ang)

        r_i = lax.broadcasted_iota(jnp.int32, (C, C), 0)
        c_i = lax.broadcasted_iota(jnp.int32, (C, C), 1)
        c_even = lax.rem(c_i, 2) == 0
        R = jnp.where(c_even & (r_i == c_i + 1), -1.0,
                      jnp.where((~c_even) & (r_i == c_i - 1), 1.0, 0.0)
                      ).astype(jnp.float32)

        def rope(t):
            t_r = jnp.dot(t, R, preferred_element_type=jnp.float32)
            return t * cos_t + t_r * sin_t

        for b in range(B):
            xb = x_ref[b]
            kb = rope(jnp.dot(xb, wk_ref[:, :],
                              preferred_element_type=jnp.float32))
            vb = jnp.dot(xb, wv_ref[:, :],
                         preferred_element_type=jnp.float32)
            kv_all[0, b, :, pl.ds(0, C)] = kb.astype(jnp.bfloat16)
            kv_all[0, b, :, pl.ds(C, C)] = vb.astype(jnp.bfloat16)

        def rcopy(src_slot, dst_slot, ssem, rsem, target):
            return pltpu.make_async_remote_copy(
                src_ref=kv_all.at[src_slot],
                dst_ref=kv_all.at[dst_slot],
                send_sem=ssem, recv_sem=rsem,
                device_id=(target,),
                device_id_type=pl.DeviceIdType.MESH,
            )

        q_heads = [[None] * HQ for _ in range(B)]
        m_st = [[None] * HQ for _ in range(B)]
        l_st = [[None] * HQ for _ in range(B)]
        acc_st = [[None] * HQ for _ in range(B)]

        def consume(slot, first=False):
            for b in range(B):
                for hd in range(HQ):
                    kc = kv_all[slot, b, :, pl.ds(hd * DH, DH)]
                    vc = kv_all[slot, b, :, pl.ds(C + hd * DH, DH)]
                    s = lax.dot_general(
                        q_heads[b][hd], kc, (((1,), (1,)), ((), ())),
                        preferred_element_type=jnp.float32) * SCALE
                    mc = jnp.max(s, axis=1, keepdims=True)
                    if first:
                        m_new = mc
                        p = jnp.exp(s - m_new)
                        l_new = jnp.sum(p, axis=1, keepdims=True)
                        a_new = jnp.dot(p.astype(jnp.bfloat16), vc,
                                        preferred_element_type=jnp.float32)
                    else:
                        m_new = jnp.maximum(m_st[b][hd], mc)
                        alpha = jnp.exp(m_st[b][hd] - m_new)
                        p = jnp.exp(s - m_new)
                        l_new = l_st[b][hd] * alpha + jnp.sum(
                            p, axis=1, keepdims=True)
                        a_new = acc_st[b][hd] * alpha + jnp.dot(
                            p.astype(jnp.bfloat16), vc,
                            preferred_element_type=jnp.float32)
                    m_st[b][hd] = m_new
                    l_st[b][hd] = l_new
                    acc_st[b][hd] = a_new

        for h in range(NR):
            r = rcopy(h, h + 1, ssem_r.at[h % 2], rsem_r.at[h], right)
            r.start()
            l = None
            if h < NL:
                l = rcopy(8 + h if h else 0, 9 + h,
                          ssem_l.at[h % 2], rsem_l.at[h], left)
                l.start()

            if h == 0:
                wq_bf = wq_ref[:, :].astype(jnp.bfloat16)
                for b in range(B):
                    qb = rope(jnp.dot(x_ref[b].astype(jnp.bfloat16), wq_bf,
                                      preferred_element_type=jnp.float32))
                    qb = qb.astype(jnp.bfloat16)
                    for hd in range(HQ):
                        q_heads[b][hd] = qb[:, hd * DH:(hd + 1) * DH]
                consume(0, first=True)
            else:
                consume(h)
                consume(8 + h)

            r.wait()
            if l is not None:
                l.wait()

        consume(NR)

        wo_bf = wo_ref[:, :].astype(jnp.bfloat16)
        for b in range(B):
            ctx = jnp.concatenate(
                [acc_st[b][hd] / l_st[b][hd] for hd in range(HQ)], axis=1)
            out_ref[b] = jnp.dot(ctx.astype(jnp.bfloat16), wo_bf,
                                 preferred_element_type=jnp.float32)

    return pl.pallas_call(
        body,
        out_shape=jax.ShapeDtypeStruct((B, S, D), jnp.float32),
        in_specs=[pl.BlockSpec(memory_space=pltpu.VMEM)] * 5,
        out_specs=pl.BlockSpec(memory_space=pltpu.VMEM),
        scratch_shapes=[
            pltpu.VMEM((N_DEV, B, S, KV), jnp.bfloat16),
            pltpu.SemaphoreType.DMA((2,)),
            pltpu.SemaphoreType.DMA((NR,)),
            pltpu.SemaphoreType.DMA((2,)),
            pltpu.SemaphoreType.DMA((NL,)),
        ],
        compiler_params=pltpu.CompilerParams(collective_id=0),
    )(x, Wq, Wk, Wv, Wo)


# device time: 86637 ns/iter; 1.0096x vs baseline; 1.0096x over previous
import jax
import jax.numpy as jnp
from jax import lax
from jax.experimental import pallas as pl
from jax.experimental.pallas import tpu as pltpu

N_DEV = 16
NR = 8
NL = 7
B = 2
S = 256
D = 768
HQ = 4
DH = 64
C = HQ * DH
KV = 2 * C
SCALE = 0.125


def kernel(x, Wq, Wk, Wv, Wo):
    def body(x_ref, wq_ref, wk_ref, wv_ref, wo_ref, out_ref,
             kv_all, ssem_r, rsem_r, ssem_l, rsem_l):
        my = lax.axis_index("i")
        left = lax.rem(my + N_DEV - 1, N_DEV)
        right = lax.rem(my + 1, N_DEV)

        barrier = pltpu.get_barrier_semaphore()
        for nbr in (left, right):
            pl.semaphore_signal(barrier, inc=1, device_id=(nbr,),
                                device_id_type=pl.DeviceIdType.MESH)
        pl.semaphore_wait(barrier, 2)

        row = lax.broadcasted_iota(jnp.int32, (S, C), 0).astype(jnp.float32)
        col = lax.broadcasted_iota(jnp.int32, (S, C), 1)
        j = (lax.rem(col, DH) // 2).astype(jnp.float32)
        inv = jnp.exp(j * (-2.0 * jnp.log(10000.0) / DH))
        pos = row + my.astype(jnp.float32) * float(S)
        ang = pos * inv
        cos_t = jnp.cos(ang)
        sin_t = jnp.sin(ang)

        r_i = lax.broadcasted_iota(jnp.int32, (C, C), 0)
        c_i = lax.broadcasted_iota(jnp.int32, (C, C), 1)
        c_even = lax.rem(c_i, 2) == 0
        R = jnp.where(c_even & (r_i == c_i + 1), -1.0,
                      jnp.where((~c_even) & (r_i == c_i - 1), 1.0, 0.0)
                      ).astype(jnp.float32)

        def rope(t):
            t_r = jnp.dot(t, R, preferred_element_type=jnp.float32)
            return t * cos_t + t_r * sin_t

        def rcopy(src_slot, dst_slot, ssem, rsem, target):
            return pltpu.make_async_remote_copy(
                src_ref=kv_all.at[src_slot],
                dst_ref=kv_all.at[dst_slot],
                send_sem=ssem, recv_sem=rsem,
                device_id=(target,),
                device_id_type=pl.DeviceIdType.MESH,
            )

        q_heads = [[None] * HQ for _ in range(B)]
        m_st = [[None] * HQ for _ in range(B)]
        l_st = [[None] * HQ for _ in range(B)]
        acc_st = [[None] * HQ for _ in range(B)]

        def consume(slot, first=False):
            for b in range(B):
                for hd in range(HQ):
                    kc = kv_all[slot, b, :, pl.ds(hd * DH, DH)].astype(
                        jnp.float32)
                    vc = kv_all[slot, b, :, pl.ds(C + hd * DH, DH)].astype(
                        jnp.float32)
                    s = lax.dot_general(
                        q_heads[b][hd], kc, (((1,), (1,)), ((), ())),
                        preferred_element_type=jnp.float32) * SCALE
                    mc = jnp.max(s, axis=1, keepdims=True)
                    if first:
                        m_new = mc
                        p = jnp.exp(s - m_new)
                        l_new = jnp.sum(p, axis=1, keepdims=True)
                        a_new = jnp.dot(p, vc,
                                        preferred_element_type=jnp.float32)
                    else:
                        m_new = jnp.maximum(m_st[b][hd], mc)
                        alpha = jnp.exp(m_st[b][hd] - m_new)
                        p = jnp.exp(s - m_new)
                        l_new = l_st[b][hd] * alpha + jnp.sum(
                            p, axis=1, keepdims=True)
                        a_new = acc_st[b][hd] * alpha + jnp.dot(
                            p, vc, preferred_element_type=jnp.float32)
                    m_st[b][hd] = m_new
                    l_st[b][hd] = l_new
                    acc_st[b][hd] = a_new

        h0_descs = []
        for b in range(B):
            xb = x_ref[b]
            kb = rope(jnp.dot(xb, wk_ref[:, :],
                              preferred_element_type=jnp.float32))
            vb = jnp.dot(xb, wv_ref[:, :],
                         preferred_element_type=jnp.float32)
            kv_all[0, b, :, pl.ds(0, C)] = kb.astype(jnp.bfloat16)
            kv_all[0, b, :, pl.ds(C, C)] = vb.astype(jnp.bfloat16)
            r0 = pltpu.make_async_remote_copy(
                src_ref=kv_all.at[0, b], dst_ref=kv_all.at[1, b],
                send_sem=ssem_r.at[b], recv_sem=rsem_r.at[NR if b else 0],
                device_id=(right,), device_id_type=pl.DeviceIdType.MESH)
            l0 = pltpu.make_async_remote_copy(
                src_ref=kv_all.at[0, b], dst_ref=kv_all.at[9, b],
                send_sem=ssem_l.at[b], recv_sem=rsem_l.at[NL if b else 0],
                device_id=(left,), device_id_type=pl.DeviceIdType.MESH)
            r0.start()
            l0.start()
            h0_descs += [r0, l0]

        for b in range(B):
            qb = rope(jnp.dot(x_ref[b], wq_ref[:, :],
                              preferred_element_type=jnp.float32))
            for hd in range(HQ):
                q_heads[b][hd] = qb[:, hd * DH:(hd + 1) * DH]
        consume(0, first=True)
        for d in h0_descs:
            d.wait()

        for h in range(1, NR):
            r = rcopy(h, h + 1, ssem_r.at[h % 2], rsem_r.at[h], right)
            r.start()
            l = None
            if h < NL:
                l = rcopy(8 + h, 9 + h,
                          ssem_l.at[h % 2], rsem_l.at[h], left)
                l.start()

            consume(h)
            consume(8 + h)

            r.wait()
            if l is not None:
                l.wait()

        consume(NR)

        for b in range(B):
            ctx = jnp.concatenate(
                [acc_st[b][hd] / l_st[b][hd] for hd in range(HQ)], axis=1)
            out_ref[b] = jnp.dot(ctx, wo_ref[:, :],
                                 preferred_element_type=jnp.float32)

    return pl.pallas_call(
        body,
        out_shape=jax.ShapeDtypeStruct((B, S, D), jnp.float32),
        in_specs=[pl.BlockSpec(memory_space=pltpu.VMEM)] * 5,
        out_specs=pl.BlockSpec(memory_space=pltpu.VMEM),
        scratch_shapes=[
            pltpu.VMEM((N_DEV, B, S, KV), jnp.bfloat16),
            pltpu.SemaphoreType.DMA((2,)),
            pltpu.SemaphoreType.DMA((NR + 1,)),
            pltpu.SemaphoreType.DMA((2,)),
            pltpu.SemaphoreType.DMA((NL + 1,)),
        ],
        compiler_params=pltpu.CompilerParams(collective_id=0),
    )(x, Wq, Wk, Wv, Wo)


# device time: 74546 ns/iter; 1.1733x vs baseline; 1.1622x over previous
import jax
import jax.numpy as jnp
from jax import lax
from jax.experimental import pallas as pl
from jax.experimental.pallas import tpu as pltpu

N_DEV = 16
NR = 8
NL = 7
B = 2
S = 256
D = 768
HQ = 4
DH = 64
C = HQ * DH
KV = 2 * C
SCALE = 0.125


def kernel(x, Wq, Wk, Wv, Wo):
    SUCC = [4, 2, 6, 0, 8, 1, 10, 3, 12, 5, 14, 7, 13, 9, 15, 11]
    PRED = [3, 5, 1, 7, 0, 9, 2, 11, 4, 13, 6, 15, 8, 12, 10, 14]

    def body(x_ref, wq_ref, wk_ref, wv_ref, wo_ref, out_ref,
             kv_all, ssem_r, rsem_r, ssem_l, rsem_l):
        my = lax.axis_index("i")
        right = my * 0
        left = my * 0
        for p in range(N_DEV):
            right = jnp.where(my == p, SUCC[p], right)
            left = jnp.where(my == p, PRED[p], left)

        barrier = pltpu.get_barrier_semaphore()
        for nbr in (left, right):
            pl.semaphore_signal(barrier, inc=1, device_id=(nbr,),
                                device_id_type=pl.DeviceIdType.MESH)
        pl.semaphore_wait(barrier, 2)

        row = lax.broadcasted_iota(jnp.int32, (S, C), 0).astype(jnp.float32)
        col = lax.broadcasted_iota(jnp.int32, (S, C), 1)
        j = (lax.rem(col, DH) // 2).astype(jnp.float32)
        inv = jnp.exp(j * (-2.0 * jnp.log(10000.0) / DH))
        pos = row + my.astype(jnp.float32) * float(S)
        ang = pos * inv
        cos_t = jnp.cos(ang)
        sin_t = jnp.sin(ang)

        r_i = lax.broadcasted_iota(jnp.int32, (C, C), 0)
        c_i = lax.broadcasted_iota(jnp.int32, (C, C), 1)
        c_even = lax.rem(c_i, 2) == 0
        R = jnp.where(c_even & (r_i == c_i + 1), -1.0,
                      jnp.where((~c_even) & (r_i == c_i - 1), 1.0, 0.0)
                      ).astype(jnp.float32)

        def rope(t):
            t_r = jnp.dot(t, R, preferred_element_type=jnp.float32)
            return t * cos_t + t_r * sin_t

        def rcopy(src_slot, dst_slot, ssem, rsem, target):
            return pltpu.make_async_remote_copy(
                src_ref=kv_all.at[src_slot],
                dst_ref=kv_all.at[dst_slot],
                send_sem=ssem, recv_sem=rsem,
                device_id=(target,),
                device_id_type=pl.DeviceIdType.MESH,
            )

        q_heads = [[None] * HQ for _ in range(B)]
        m_st = [[None] * HQ for _ in range(B)]
        l_st = [[None] * HQ for _ in range(B)]
        acc_st = [[None] * HQ for _ in range(B)]

        def consume(slot, first=False):
            for b in range(B):
                for hd in range(HQ):
                    kc = kv_all[slot, b, :, pl.ds(hd * DH, DH)].astype(
                        jnp.float32)
                    vc = kv_all[slot, b, :, pl.ds(C + hd * DH, DH)].astype(
                        jnp.float32)
                    s = lax.dot_general(
                        q_heads[b][hd], kc, (((1,), (1,)), ((), ())),
                        preferred_element_type=jnp.float32) * SCALE
                    mc = jnp.max(s, axis=1, keepdims=True)
                    if first:
                        m_new = mc
                        p = jnp.exp(s - m_new)
                        l_new = jnp.sum(p, axis=1, keepdims=True)
                        a_new = jnp.dot(p, vc,
                                        preferred_element_type=jnp.float32)
                    else:
                        m_new = jnp.maximum(m_st[b][hd], mc)
                        alpha = jnp.exp(m_st[b][hd] - m_new)
                        p = jnp.exp(s - m_new)
                        l_new = l_st[b][hd] * alpha + jnp.sum(
                            p, axis=1, keepdims=True)
                        a_new = acc_st[b][hd] * alpha + jnp.dot(
                            p, vc, preferred_element_type=jnp.float32)
                    m_st[b][hd] = m_new
                    l_st[b][hd] = l_new
                    acc_st[b][hd] = a_new

        h0_descs = []
        for b in range(B):
            xb = x_ref[b]
            kb = rope(jnp.dot(xb, wk_ref[:, :],
                              preferred_element_type=jnp.float32))
            vb = jnp.dot(xb, wv_ref[:, :],
                         preferred_element_type=jnp.float32)
            kv_all[0, b, :, pl.ds(0, C)] = kb.astype(jnp.bfloat16)
            kv_all[0, b, :, pl.ds(C, C)] = vb.astype(jnp.bfloat16)
            r0 = pltpu.make_async_remote_copy(
                src_ref=kv_all.at[0, b], dst_ref=kv_all.at[1, b],
                send_sem=ssem_r.at[b], recv_sem=rsem_r.at[NR if b else 0],
                device_id=(right,), device_id_type=pl.DeviceIdType.MESH)
            l0 = pltpu.make_async_remote_copy(
                src_ref=kv_all.at[0, b], dst_ref=kv_all.at[9, b],
                send_sem=ssem_l.at[b], recv_sem=rsem_l.at[NL if b else 0],
                device_id=(left,), device_id_type=pl.DeviceIdType.MESH)
            r0.start()
            l0.start()
            h0_descs += [r0, l0]

        for b in range(B):
            qb = rope(jnp.dot(x_ref[b], wq_ref[:, :],
                              preferred_element_type=jnp.float32))
            for hd in range(HQ):
                q_heads[b][hd] = qb[:, hd * DH:(hd + 1) * DH]
        consume(0, first=True)
        for d in h0_descs:
            d.wait()

        for h in range(1, NR):
            r = rcopy(h, h + 1, ssem_r.at[h % 2], rsem_r.at[h], right)
            r.start()
            l = None
            if h < NL:
                l = rcopy(8 + h, 9 + h,
                          ssem_l.at[h % 2], rsem_l.at[h], left)
                l.start()

            consume(h)
            consume(8 + h)

            r.wait()
            if l is not None:
                l.wait()

        consume(NR)

        for b in range(B):
            ctx = jnp.concatenate(
                [acc_st[b][hd] / l_st[b][hd] for hd in range(HQ)], axis=1)
            out_ref[b] = jnp.dot(ctx, wo_ref[:, :],
                                 preferred_element_type=jnp.float32)

    return pl.pallas_call(
        body,
        out_shape=jax.ShapeDtypeStruct((B, S, D), jnp.float32),
        in_specs=[pl.BlockSpec(memory_space=pltpu.VMEM)] * 5,
        out_specs=pl.BlockSpec(memory_space=pltpu.VMEM),
        scratch_shapes=[
            pltpu.VMEM((N_DEV, B, S, KV), jnp.bfloat16),
            pltpu.SemaphoreType.DMA((2,)),
            pltpu.SemaphoreType.DMA((NR + 1,)),
            pltpu.SemaphoreType.DMA((2,)),
            pltpu.SemaphoreType.DMA((NL + 1,)),
        ],
        compiler_params=pltpu.CompilerParams(collective_id=0),
    )(x, Wq, Wk, Wv, Wo)
